# baseline (speedup 1.0000x reference)
import functools

import jax
import jax.numpy as jnp
from jax import lax
from jax.experimental import pallas as pl
from jax.experimental.pallas import tpu as pltpu
from jax.experimental.pallas import tpu_sc as plsc

B = 16384
CTX = 20
D = 64
NC = 2
NS = 16
NW = NC * NS
BPW = B // NW
CB = 32
NCHUNK = BPW // CB
PAIRS = CB * CTX
NGRP = PAIRS // 16
IDXW = 128
NIDX = PAIRS // IDXW


def _sc_skipgram(target_flat, ctx_flat, target_table, context_table):
    mesh = plsc.VectorSubcoreMesh(core_axis_name="c", subcore_axis_name="s")

    @functools.partial(
        pl.kernel,
        mesh=mesh,
        compiler_params=pltpu.CompilerParams(
            needs_layout_passes=False, use_tc_tiling_on_sc=False),
        out_type=jax.ShapeDtypeStruct((B * CTX,), jnp.float32),
        scratch_types=[
            pltpu.VMEM((CB,), jnp.int32),
            pltpu.VMEM((PAIRS,), jnp.int32),
            pltpu.VMEM((CB, D), jnp.bfloat16),
            pltpu.VMEM((PAIRS, D), jnp.bfloat16),
            pltpu.VMEM((PAIRS,), jnp.float32),
            pltpu.SemaphoreType.DMA,
        ],
    )
    def k(tgt_hbm, ctx_hbm, ttab_hbm, ctab_hbm, out_hbm,
          tidx, cidx, te, ce, ob, sem):
        wid = lax.axis_index("s") * NC + lax.axis_index("c")
        lanes = lax.iota(jnp.int32, 16)

        def chunk_body(c, carry):
            base_b = wid * BPW + c * CB
            pltpu.sync_copy(tgt_hbm.at[pl.ds(base_b, CB)], tidx)
            coff = wid * BPW * CTX + c * PAIRS
            pltpu.sync_copy(ctx_hbm.at[pl.ds(coff, PAIRS)], cidx)

            cp_t = pltpu.async_copy(ttab_hbm.at[tidx], te, sem)
            cps = [
                pltpu.async_copy(ctab_hbm.at[cidx.at[pl.ds(kk * IDXW, IDXW)]],
                                 ce.at[pl.ds(kk * IDXW, IDXW)], sem)
                for kk in range(NIDX)
            ]
            cp_t.wait()
            for cp in cps:
                cp.wait()

            def sg_body(sg, carry2):
                b0 = sg * 4
                accs = [jnp.zeros((16,), jnp.float32) for _ in range(5)]
                for boff in range(4):
                    b = b0 + boff
                    t0, t1 = plsc.unpack(te[b, pl.ds(0, 32)],
                                         format=plsc.PackFormat.INTERLEAVED,
                                         preferred_element_type=jnp.float32)
                    t2, t3 = plsc.unpack(te[b, pl.ds(32, 32)],
                                         format=plsc.PackFormat.INTERLEAVED,
                                         preferred_element_type=jnp.float32)
                    for j in range(CTX):
                        p = boff * CTX + j
                        row = b * CTX + j
                        c0, c1 = plsc.unpack(ce[row, pl.ds(0, 32)],
                                             format=plsc.PackFormat.INTERLEAVED,
                                             preferred_element_type=jnp.float32)
                        c2, c3 = plsc.unpack(ce[row, pl.ds(32, 32)],
                                             format=plsc.PackFormat.INTERLEAVED,
                                             preferred_element_type=jnp.float32)
                        s = t0 * c0 + t1 * c1 + t2 * c2 + t3 * c3
                        v, l = divmod(p, 16)
                        accs[v] = jnp.where(lanes == l, jnp.sum(s), accs[v])
                for v in range(5):
                    ob[pl.ds(sg * 80 + v * 16, 16)] = accs[v]
                return carry2

            lax.fori_loop(0, CB // 4, sg_body, 0, unroll=False)

            out0 = wid * BPW * CTX + c * PAIRS
            pltpu.sync_copy(ob, out_hbm.at[pl.ds(out0, PAIRS)])
            return carry

        lax.fori_loop(0, NCHUNK, chunk_body, 0, unroll=False)

    return k(target_flat, ctx_flat, target_table, context_table)


def kernel(target, context, target_table, context_table):
    target_flat = target.reshape(B)
    ctx_flat = context.reshape(B * CTX)
    out_flat = _sc_skipgram(target_flat, ctx_flat,
                            target_table.astype(jnp.bfloat16),
                            context_table.astype(jnp.bfloat16))
    return out_flat.reshape(B, CTX)


# speedup vs baseline: 1.3933x; 1.3933x over previous
import functools

import jax
import jax.numpy as jnp
from jax import lax
from jax.experimental import pallas as pl
from jax.experimental.pallas import tpu as pltpu
from jax.experimental.pallas import tpu_sc as plsc

B = 16384
CTX = 20
D = 64
VOCAB = 1000000
NC = 2
NS = 16
NW = NC * NS
BPW = B // NW
CB = 32
NCHUNK = BPW // CB
PAIRS = CB * CTX
NGRP = PAIRS // 16
IDXW = 128
NIDX = PAIRS // IDXW

TCOLS = 2048


def _tc_relayout(table_t):
    grid = (VOCAB + TCOLS - 1) // TCOLS

    def body(in_ref, out_ref):
        x = in_ref[...]
        out_ref[:, 0:D] = jnp.swapaxes(x, 0, 1)

    return pl.pallas_call(
        body,
        grid=(grid,),
        in_specs=[pl.BlockSpec((D, TCOLS), lambda i: (0, i))],
        out_specs=pl.BlockSpec((TCOLS, 128), lambda i: (i, 0)),
        out_shape=jax.ShapeDtypeStruct((VOCAB, 128), jnp.float32),
    )(table_t)


def _sc_skipgram(target_flat, ctx_flat, ttab, ctab):
    mesh = plsc.VectorSubcoreMesh(core_axis_name="c", subcore_axis_name="s")

    @functools.partial(
        pl.kernel,
        mesh=mesh,
        compiler_params=pltpu.CompilerParams(
            needs_layout_passes=False, use_tc_tiling_on_sc=True),
        out_type=jax.ShapeDtypeStruct((B * CTX,), jnp.float32),
        scratch_types=[
            pltpu.VMEM((CB,), jnp.int32),
            pltpu.VMEM((PAIRS,), jnp.int32),
            pltpu.VMEM((CB, 128), jnp.float32),
            pltpu.VMEM((PAIRS, 128), jnp.float32),
            pltpu.VMEM((PAIRS,), jnp.float32),
            pltpu.SemaphoreType.DMA,
        ],
    )
    def k(tgt_hbm, ctx_hbm, ttab_hbm, ctab_hbm, out_hbm,
          tidx, cidx, te, ce, ob, sem):
        wid = lax.axis_index("s") * NC + lax.axis_index("c")
        lanes = lax.iota(jnp.int32, 16)

        def chunk_body(c, carry):
            base_b = wid * BPW + c * CB
            pltpu.sync_copy(tgt_hbm.at[pl.ds(base_b, CB)], tidx)
            coff = wid * BPW * CTX + c * PAIRS
            pltpu.sync_copy(ctx_hbm.at[pl.ds(coff, PAIRS)], cidx)

            cp_t = pltpu.async_copy(ttab_hbm.at[tidx], te, sem)
            cps = [
                pltpu.async_copy(ctab_hbm.at[cidx.at[pl.ds(kk * IDXW, IDXW)]],
                                 ce.at[pl.ds(kk * IDXW, IDXW)], sem)
                for kk in range(NIDX)
            ]
            cp_t.wait()
            for cp in cps:
                cp.wait()

            def sg_body(sg, carry2):
                b0 = sg * 4
                accs = [jnp.zeros((16,), jnp.float32) for _ in range(5)]
                for boff in range(4):
                    b = b0 + boff
                    t0 = te[b, pl.ds(0, 16)]
                    t1 = te[b, pl.ds(16, 16)]
                    t2 = te[b, pl.ds(32, 16)]
                    t3 = te[b, pl.ds(48, 16)]
                    for j in range(CTX):
                        p = boff * CTX + j
                        row = b * CTX + j
                        s = (t0 * ce[row, pl.ds(0, 16)]
                             + t1 * ce[row, pl.ds(16, 16)]
                             + t2 * ce[row, pl.ds(32, 16)]
                             + t3 * ce[row, pl.ds(48, 16)])
                        v, l = divmod(p, 16)
                        accs[v] = jnp.where(lanes == l, jnp.sum(s), accs[v])
                for v in range(5):
                    ob[pl.ds(sg * 80 + v * 16, 16)] = accs[v]
                return carry2

            lax.fori_loop(0, CB // 4, sg_body, 0, unroll=False)

            out0 = wid * BPW * CTX + c * PAIRS
            pltpu.sync_copy(ob, out_hbm.at[pl.ds(out0, PAIRS)])
            return carry

        lax.fori_loop(0, NCHUNK, chunk_body, 0, unroll=False)

    return k(target_flat, ctx_flat, ttab, ctab)


def kernel(target, context, target_table, context_table):
    target_flat = target.reshape(B)
    ctx_flat = context.reshape(B * CTX)
    ttab = _tc_relayout(target_table.T)
    ctab = _tc_relayout(context_table.T)
    out_flat = _sc_skipgram(target_flat, ctx_flat, ttab, ctab)
    return out_flat.reshape(B, CTX)
